# baseline (speedup 1.0000x reference)
import functools

import jax
import jax.numpy as jnp
from jax.experimental import pallas as pl
from jax.experimental.pallas import tpu as pltpu

_BF = jnp.bfloat16
_VMEM = 64 * 1024 * 1024


def _enc_conv_body(x_ref, w_ref, b_ref, o_ref, *, oh, ow):
    x = x_ref[...]
    bb = x.shape[0]
    c2 = x.shape[-1]
    taps = []
    for kh in range(4):
        qi, hp = kh // 2, kh % 2
        for dj in range(2):
            t = x[:, qi:qi + oh, hp:hp + 1, dj:dj + ow, :]
            taps.append(t.reshape(bb, oh, ow, c2))
    a = jnp.concatenate(taps, axis=-1)
    a2 = a.reshape(bb * oh * ow, a.shape[-1])
    acc = jnp.dot(a2, w_ref[...], preferred_element_type=jnp.float32)
    acc = jnp.maximum(acc + b_ref[...], 0.0)
    o_ref[...] = acc.reshape(bb, oh, ow, acc.shape[-1]).astype(o_ref.dtype)


def _enc_conv(xr, wm, b, oh, ow, bb):
    B = xr.shape[0]
    bb = min(bb, B)
    cout = wm.shape[1]
    return pl.pallas_call(
        functools.partial(_enc_conv_body, oh=oh, ow=ow),
        out_shape=jax.ShapeDtypeStruct((B, oh, ow, cout), _BF),
        grid=(B // bb,),
        in_specs=[
            pl.BlockSpec((bb,) + xr.shape[1:], lambda i: (i, 0, 0, 0, 0)),
            pl.BlockSpec(wm.shape, lambda i: (0, 0)),
            pl.BlockSpec((1, cout), lambda i: (0, 0)),
        ],
        out_specs=pl.BlockSpec((bb, oh, ow, cout), lambda i: (i, 0, 0, 0)),
        compiler_params=pltpu.CompilerParams(
            dimension_semantics=("parallel",), vmem_limit_bytes=_VMEM),
    )(xr, wm, b.reshape(1, cout))


def _gemm_body(a_ref, w_ref, b_ref, o_ref):
    acc = jnp.dot(a_ref[...], w_ref[...], preferred_element_type=jnp.float32)
    o_ref[...] = jnp.maximum(acc + b_ref[...], 0.0).astype(o_ref.dtype)


def _gemm_relu(a, wm, b, tm):
    M = a.shape[0]
    tm = min(tm, M)
    n = wm.shape[1]
    return pl.pallas_call(
        _gemm_body,
        out_shape=jax.ShapeDtypeStruct((M, n), _BF),
        grid=(M // tm,),
        in_specs=[
            pl.BlockSpec((tm, a.shape[1]), lambda i: (i, 0)),
            pl.BlockSpec(wm.shape, lambda i: (0, 0)),
            pl.BlockSpec((1, n), lambda i: (0, 0)),
        ],
        out_specs=pl.BlockSpec((tm, n), lambda i: (i, 0)),
        compiler_params=pltpu.CompilerParams(
            dimension_semantics=("parallel",), vmem_limit_bytes=_VMEM),
    )(a, wm, b.reshape(1, n))


def _dec_body(x_ref, w_ref, b_ref, o_ref, *, h, w, relu, shuffle):
    x = x_ref[...]
    taps = [x[:, dh:dh + h + 1, dw:dw + w + 1, :]
            for dh in (0, 1) for dw in (0, 1)]
    a = jnp.concatenate(taps, axis=-1)
    bb = a.shape[0]
    a2 = a.reshape(bb * (h + 1) * (w + 1), a.shape[-1])
    acc = jnp.dot(a2, w_ref[...], preferred_element_type=jnp.float32)
    acc = acc + b_ref[...]
    if relu:
        acc = jnp.maximum(acc, 0.0)
    n4 = acc.shape[-1]
    c = n4 // 4
    if not shuffle:
        o_ref[...] = acc.reshape(bb, h + 1, w + 1, n4).astype(o_ref.dtype)
        return
    y = acc.astype(o_ref.dtype).reshape(bb, h + 1, w + 1, n4)
    y0, y1 = y[..., :2 * c], y[..., 2 * c:]
    r = jnp.stack([y1[:, 0:h], y0[:, 1:h + 1]],
                  axis=2).reshape(bb, 2 * h, w + 1, 2 * c)
    rp0, rp1 = r[..., :c], r[..., c:]
    o_ref[...] = jnp.concatenate(
        [rp1[:, :, 0:w, :], rp0[:, :, 1:w + 1, :]], axis=-1)


def _dec_conv(xp, wm, b, h, w, bb, *, relu, shuffle, out_dtype):
    B = xp.shape[0]
    bb = min(bb, B)
    n4 = wm.shape[1]
    oshape = ((B, 2 * h, w, n4 // 2) if shuffle
              else (B, h + 1, w + 1, n4))
    blk = (bb,) + oshape[1:]
    return pl.pallas_call(
        functools.partial(_dec_body, h=h, w=w, relu=relu, shuffle=shuffle),
        out_shape=jax.ShapeDtypeStruct(oshape, out_dtype),
        grid=(B // bb,),
        in_specs=[
            pl.BlockSpec((bb,) + xp.shape[1:], lambda i: (i, 0, 0, 0)),
            pl.BlockSpec(wm.shape, lambda i: (0, 0)),
            pl.BlockSpec((1, n4), lambda i: (0, 0)),
        ],
        out_specs=pl.BlockSpec(blk, lambda i: (i, 0, 0, 0)),
        compiler_params=pltpu.CompilerParams(
            dimension_semantics=("parallel",), vmem_limit_bytes=_VMEM),
    )(xp, wm, b.reshape(1, n4))


def _latent_body(h_ref, noise_ref, wfc_ref, bfc_ref, wml_ref, bml_ref,
                 wlat_ref, blat_ref, wdec_ref, bdec_ref,
                 d_ref, z_ref, mean_ref, logvar_ref, *, nl):
    h1 = jnp.maximum(
        jnp.dot(h_ref[...], wfc_ref[...], preferred_element_type=jnp.float32)
        + bfc_ref[...], 0.0)
    ml = (jnp.dot(h1.astype(_BF), wml_ref[...],
                  preferred_element_type=jnp.float32) + bml_ref[...])
    mean = ml[:, :nl]
    logvar = ml[:, nl:]
    z = noise_ref[...] * jnp.exp(0.5 * logvar) + mean
    d1 = jnp.maximum(
        jnp.dot(z.astype(_BF), wlat_ref[...],
                preferred_element_type=jnp.float32) + blat_ref[...], 0.0)
    d2 = jnp.maximum(
        jnp.dot(d1.astype(_BF), wdec_ref[...],
                preferred_element_type=jnp.float32) + bdec_ref[...], 0.0)
    d_ref[...] = d2.astype(d_ref.dtype)
    z_ref[...] = z
    mean_ref[...] = mean
    logvar_ref[...] = logvar


def _latent(h, noise, wfc, bfc, wml, bml, wlat, blat, wdec, bdec, nl):
    B = h.shape[0]
    bb = B // 2 if B % 2 == 0 else B
    full = lambda arr: pl.BlockSpec(arr.shape, lambda i: (0, 0))
    out_shapes = (
        jax.ShapeDtypeStruct((B, 1024), _BF),
        jax.ShapeDtypeStruct((B, nl), jnp.float32),
        jax.ShapeDtypeStruct((B, nl), jnp.float32),
        jax.ShapeDtypeStruct((B, nl), jnp.float32),
    )
    return pl.pallas_call(
        functools.partial(_latent_body, nl=nl),
        out_shape=out_shapes,
        grid=(B // bb,),
        in_specs=[
            pl.BlockSpec((bb, h.shape[1]), lambda i: (i, 0)),
            pl.BlockSpec((bb, nl), lambda i: (i, 0)),
            full(wfc), full(bfc), full(wml), full(bml),
            full(wlat), full(blat), full(wdec), full(bdec),
        ],
        out_specs=(
            pl.BlockSpec((bb, 1024), lambda i: (i, 0)),
            pl.BlockSpec((bb, nl), lambda i: (i, 0)),
            pl.BlockSpec((bb, nl), lambda i: (i, 0)),
            pl.BlockSpec((bb, nl), lambda i: (i, 0)),
        ),
        compiler_params=pltpu.CompilerParams(
            dimension_semantics=("parallel",), vmem_limit_bytes=_VMEM),
    )(h, noise, wfc, bfc, wml, bml, wlat, blat, wdec, bdec)


def _conv_wm(wt):
    return jnp.transpose(wt, (2, 3, 1, 0)).reshape(-1, wt.shape[0]).astype(_BF)


def _phase_wm(wt):
    sel = ((2, 0), (3, 1))
    rows = []
    for dh in range(2):
        for dw in range(2):
            cols = [wt[:, :, sel[ph][dh], sel[pw][dw]]
                    for ph in range(2) for pw in range(2)]
            rows.append(jnp.concatenate(cols, axis=1))
    return jnp.concatenate(rows, axis=0).astype(_BF)


def _pair(y):
    B, H, W, C = y.shape
    yp = jnp.pad(y, ((0, 0), (1, 1), (1, 1), (0, 0)))
    return yp.reshape(B, (H + 2) // 2, 2, (W + 2) // 2, 2 * C)


def _halo(y):
    return jnp.pad(y, ((0, 0), (1, 1), (1, 1), (0, 0)))


def kernel(x, noise,
           conv1_w, conv1_b, conv2_w, conv2_b, conv3_w, conv3_b,
           conv4_w, conv4_b, fc_w, fc_b, fc_mean_w, fc_mean_b,
           fc_logvar_w, fc_logvar_b, fc_latent_w, fc_latent_b,
           fc_dec_w, fc_dec_b, deconv1_w, deconv1_b, deconv2_w, deconv2_b,
           deconv3_w, deconv3_b, deconv4_w, deconv4_b):
    B = x.shape[0]
    nl = noise.shape[1]

    t = jnp.transpose(x, (0, 2, 3, 1))
    xp = jnp.pad(t, ((0, 0), (1, 1), (1, 1), (0, 0)))
    cols = [xp[:, kh:kh + 64:2, kw:kw + 64:2, :]
            for kh in range(4) for kw in range(4)]
    a1 = jnp.concatenate(cols, axis=-1).astype(_BF).reshape(B * 1024, 48)
    h = _gemm_relu(a1, _conv_wm(conv1_w), conv1_b, 2048)
    h = h.reshape(B, 32, 32, 32)
    h = _enc_conv(_pair(h), _conv_wm(conv2_w), conv2_b, 16, 16, 16)
    h = _enc_conv(_pair(h), _conv_wm(conv3_w), conv3_b, 8, 8, 32)
    h = _enc_conv(_pair(h), _conv_wm(conv4_w), conv4_b, 4, 4, 64)
    hflat = h.reshape(B, 1024)

    wfc_p = (fc_w.T.reshape(64, 4, 4, 256).transpose(1, 2, 0, 3)
             .reshape(1024, 256).astype(_BF))
    wml = jnp.concatenate([fc_mean_w.T, fc_logvar_w.T], axis=1).astype(_BF)
    bml = jnp.concatenate([fc_mean_b, fc_logvar_b]).reshape(1, -1)
    wdec_p = (fc_dec_w.T.reshape(256, 64, 4, 4).transpose(0, 2, 3, 1)
              .reshape(256, 1024).astype(_BF))
    bdec_p = fc_dec_b.reshape(64, 4, 4).transpose(1, 2, 0).reshape(1, 1024)
    d, z, z_mean, z_logvar = _latent(
        hflat, noise,
        wfc_p, fc_b.reshape(1, -1), wml, bml,
        fc_latent_w.T.astype(_BF), fc_latent_b.reshape(1, -1),
        wdec_p, bdec_p, nl)

    g = d.reshape(B, 4, 4, 64)
    g = _dec_conv(_halo(g), _phase_wm(deconv1_w), jnp.tile(deconv1_b, 4),
                  4, 4, 64, relu=True, shuffle=True, out_dtype=_BF)
    g = _dec_conv(_halo(g.reshape(B, 8, 8, 64)), _phase_wm(deconv2_w),
                  jnp.tile(deconv2_b, 4),
                  8, 8, 32, relu=True, shuffle=True, out_dtype=_BF)
    g = _dec_conv(_halo(g.reshape(B, 16, 16, 32)), _phase_wm(deconv3_w),
                  jnp.tile(deconv3_b, 4),
                  16, 16, 16, relu=True, shuffle=True, out_dtype=_BF)
    y4 = _dec_conv(_halo(g.reshape(B, 32, 32, 32)), _phase_wm(deconv4_w),
                   jnp.tile(deconv4_b, 4),
                   32, 32, 8, relu=False, shuffle=False, out_dtype=_BF)
    Y = y4.reshape(B, 33, 33, 2, 2, 3)
    rec = (jnp.transpose(Y, (0, 5, 1, 3, 2, 4)).reshape(B, 3, 66, 66)
           [:, :, 1:65, 1:65])
    return rec.astype(jnp.float32), z, z_mean, z_logvar


# speedup vs baseline: 1.6485x; 1.6485x over previous
import functools

import jax
import jax.numpy as jnp
from jax.experimental import pallas as pl
from jax.experimental.pallas import tpu as pltpu

_BF = jnp.bfloat16
_VMEM = 64 * 1024 * 1024


def _enc_conv_body(x_ref, w_ref, b_ref, o_ref, *, oh, ow):
    x = x_ref[...]
    bb = x.shape[0]
    c2 = x.shape[-1]
    taps = []
    for kh in range(4):
        qi, hp = kh // 2, kh % 2
        for dj in range(2):
            t = x[:, qi:qi + oh, hp:hp + 1, dj:dj + ow, :]
            taps.append(t.reshape(bb, oh, ow, c2))
    a = jnp.concatenate(taps, axis=-1)
    a2 = a.reshape(bb * oh * ow, a.shape[-1])
    acc = jnp.dot(a2, w_ref[...], preferred_element_type=jnp.float32)
    acc = jnp.maximum(acc + b_ref[...], 0.0)
    o_ref[...] = acc.reshape(bb, oh, ow, acc.shape[-1]).astype(o_ref.dtype)


def _enc_conv(xr, wm, b, oh, ow, bb):
    B = xr.shape[0]
    bb = min(bb, B)
    cout = wm.shape[1]
    return pl.pallas_call(
        functools.partial(_enc_conv_body, oh=oh, ow=ow),
        out_shape=jax.ShapeDtypeStruct((B, oh, ow, cout), _BF),
        grid=(B // bb,),
        in_specs=[
            pl.BlockSpec((bb,) + xr.shape[1:], lambda i: (i, 0, 0, 0, 0)),
            pl.BlockSpec(wm.shape, lambda i: (0, 0)),
            pl.BlockSpec((1, cout), lambda i: (0, 0)),
        ],
        out_specs=pl.BlockSpec((bb, oh, ow, cout), lambda i: (i, 0, 0, 0)),
        compiler_params=pltpu.CompilerParams(
            dimension_semantics=("parallel",), vmem_limit_bytes=_VMEM),
    )(xr, wm, b.reshape(1, cout))


def _gemm_body(a_ref, w_ref, b_ref, o_ref):
    acc = jnp.dot(a_ref[...], w_ref[...], preferred_element_type=jnp.float32)
    o_ref[...] = jnp.maximum(acc + b_ref[...], 0.0).astype(o_ref.dtype)


def _gemm_relu(a, wm, b, tm):
    M = a.shape[0]
    tm = min(tm, M)
    n = wm.shape[1]
    return pl.pallas_call(
        _gemm_body,
        out_shape=jax.ShapeDtypeStruct((M, n), _BF),
        grid=(M // tm,),
        in_specs=[
            pl.BlockSpec((tm, a.shape[1]), lambda i: (i, 0)),
            pl.BlockSpec(wm.shape, lambda i: (0, 0)),
            pl.BlockSpec((1, n), lambda i: (0, 0)),
        ],
        out_specs=pl.BlockSpec((tm, n), lambda i: (i, 0)),
        compiler_params=pltpu.CompilerParams(
            dimension_semantics=("parallel",), vmem_limit_bytes=_VMEM),
    )(a, wm, b.reshape(1, n))


def _dec_body(x_ref, w_ref, b_ref, o_ref, *, h, w, relu, shuffle):
    x = x_ref[...]
    taps = [x[:, dh:dh + h + 1, dw:dw + w + 1, :]
            for dh in (0, 1) for dw in (0, 1)]
    a = jnp.concatenate(taps, axis=-1)
    bb = a.shape[0]
    a2 = a.reshape(bb * (h + 1) * (w + 1), a.shape[-1])
    acc = jnp.dot(a2, w_ref[...], preferred_element_type=jnp.float32)
    acc = acc + b_ref[...]
    if relu:
        acc = jnp.maximum(acc, 0.0)
    n4 = acc.shape[-1]
    c = n4 // 4
    if not shuffle:
        o_ref[...] = acc.reshape(bb, h + 1, w + 1, n4).astype(o_ref.dtype)
        return
    y = acc.astype(o_ref.dtype).reshape(bb, h + 1, w + 1, n4)
    y0, y1 = y[..., :2 * c], y[..., 2 * c:]
    r = jnp.stack([y1[:, 0:h], y0[:, 1:h + 1]],
                  axis=2).reshape(bb, 2 * h, w + 1, 2 * c)
    rp0, rp1 = r[..., :c], r[..., c:]
    o_ref[...] = jnp.concatenate(
        [rp1[:, :, 0:w, :], rp0[:, :, 1:w + 1, :]], axis=-1)


def _dec_conv(xp, wm, b, h, w, bb, *, relu, shuffle, out_dtype):
    B = xp.shape[0]
    bb = min(bb, B)
    n4 = wm.shape[1]
    oshape = ((B, 2 * h, w, n4 // 2) if shuffle
              else (B, h + 1, w + 1, n4))
    blk = (bb,) + oshape[1:]
    return pl.pallas_call(
        functools.partial(_dec_body, h=h, w=w, relu=relu, shuffle=shuffle),
        out_shape=jax.ShapeDtypeStruct(oshape, out_dtype),
        grid=(B // bb,),
        in_specs=[
            pl.BlockSpec((bb,) + xp.shape[1:], lambda i: (i, 0, 0, 0)),
            pl.BlockSpec(wm.shape, lambda i: (0, 0)),
            pl.BlockSpec((1, n4), lambda i: (0, 0)),
        ],
        out_specs=pl.BlockSpec(blk, lambda i: (i, 0, 0, 0)),
        compiler_params=pltpu.CompilerParams(
            dimension_semantics=("parallel",), vmem_limit_bytes=_VMEM),
    )(xp, wm, b.reshape(1, n4))


def _latent_body(h_ref, noise_ref, wfc_ref, bfc_ref, wml_ref, bml_ref,
                 wlat_ref, blat_ref, wdec_ref, bdec_ref,
                 d_ref, z_ref, mean_ref, logvar_ref, *, nl):
    h1 = jnp.maximum(
        jnp.dot(h_ref[...], wfc_ref[...], preferred_element_type=jnp.float32)
        + bfc_ref[...], 0.0)
    ml = (jnp.dot(h1.astype(_BF), wml_ref[...],
                  preferred_element_type=jnp.float32) + bml_ref[...])
    mean = ml[:, :nl]
    logvar = ml[:, nl:]
    z = noise_ref[...] * jnp.exp(0.5 * logvar) + mean
    d1 = jnp.maximum(
        jnp.dot(z.astype(_BF), wlat_ref[...],
                preferred_element_type=jnp.float32) + blat_ref[...], 0.0)
    d2 = jnp.maximum(
        jnp.dot(d1.astype(_BF), wdec_ref[...],
                preferred_element_type=jnp.float32) + bdec_ref[...], 0.0)
    d_ref[...] = d2.astype(d_ref.dtype)
    z_ref[...] = z
    mean_ref[...] = mean
    logvar_ref[...] = logvar


def _latent(h, noise, wfc, bfc, wml, bml, wlat, blat, wdec, bdec, nl):
    B = h.shape[0]
    bb = B // 2 if B % 2 == 0 else B
    full = lambda arr: pl.BlockSpec(arr.shape, lambda i: (0, 0))
    out_shapes = (
        jax.ShapeDtypeStruct((B, 1024), _BF),
        jax.ShapeDtypeStruct((B, nl), jnp.float32),
        jax.ShapeDtypeStruct((B, nl), jnp.float32),
        jax.ShapeDtypeStruct((B, nl), jnp.float32),
    )
    return pl.pallas_call(
        functools.partial(_latent_body, nl=nl),
        out_shape=out_shapes,
        grid=(B // bb,),
        in_specs=[
            pl.BlockSpec((bb, h.shape[1]), lambda i: (i, 0)),
            pl.BlockSpec((bb, nl), lambda i: (i, 0)),
            full(wfc), full(bfc), full(wml), full(bml),
            full(wlat), full(blat), full(wdec), full(bdec),
        ],
        out_specs=(
            pl.BlockSpec((bb, 1024), lambda i: (i, 0)),
            pl.BlockSpec((bb, nl), lambda i: (i, 0)),
            pl.BlockSpec((bb, nl), lambda i: (i, 0)),
            pl.BlockSpec((bb, nl), lambda i: (i, 0)),
        ),
        compiler_params=pltpu.CompilerParams(
            dimension_semantics=("parallel",), vmem_limit_bytes=_VMEM),
    )(h, noise, wfc, bfc, wml, bml, wlat, blat, wdec, bdec)


def _conv_wm(wt):
    return jnp.transpose(wt, (2, 3, 1, 0)).reshape(-1, wt.shape[0]).astype(_BF)


def _phase_wm(wt):
    sel = ((2, 0), (3, 1))
    rows = []
    for dh in range(2):
        for dw in range(2):
            cols = [wt[:, :, sel[ph][dh], sel[pw][dw]]
                    for ph in range(2) for pw in range(2)]
            rows.append(jnp.concatenate(cols, axis=1))
    return jnp.concatenate(rows, axis=0).astype(_BF)


def _pair(y):
    B, H, W, C = y.shape
    yp = jnp.pad(y, ((0, 0), (1, 1), (1, 1), (0, 0)))
    return yp.reshape(B, (H + 2) // 2, 2, (W + 2) // 2, 2 * C)


def _halo(y):
    return jnp.pad(y, ((0, 0), (1, 1), (1, 1), (0, 0)))


def kernel(x, noise,
           conv1_w, conv1_b, conv2_w, conv2_b, conv3_w, conv3_b,
           conv4_w, conv4_b, fc_w, fc_b, fc_mean_w, fc_mean_b,
           fc_logvar_w, fc_logvar_b, fc_latent_w, fc_latent_b,
           fc_dec_w, fc_dec_b, deconv1_w, deconv1_b, deconv2_w, deconv2_b,
           deconv3_w, deconv3_b, deconv4_w, deconv4_b):
    B = x.shape[0]
    nl = noise.shape[1]

    t = jnp.transpose(x, (0, 2, 3, 1))
    h = _pair(t.astype(_BF))
    h = _enc_conv(h, _conv_wm(conv1_w), conv1_b, 32, 32, 8)
    h = _enc_conv(_pair(h), _conv_wm(conv2_w), conv2_b, 16, 16, 16)
    h = _enc_conv(_pair(h), _conv_wm(conv3_w), conv3_b, 8, 8, 32)
    h = _enc_conv(_pair(h), _conv_wm(conv4_w), conv4_b, 4, 4, 64)
    hflat = h.reshape(B, 1024)

    wfc_p = (fc_w.T.reshape(64, 4, 4, 256).transpose(1, 2, 0, 3)
             .reshape(1024, 256).astype(_BF))
    wml = jnp.concatenate([fc_mean_w.T, fc_logvar_w.T], axis=1).astype(_BF)
    bml = jnp.concatenate([fc_mean_b, fc_logvar_b]).reshape(1, -1)
    wdec_p = (fc_dec_w.T.reshape(256, 64, 4, 4).transpose(0, 2, 3, 1)
              .reshape(256, 1024).astype(_BF))
    bdec_p = fc_dec_b.reshape(64, 4, 4).transpose(1, 2, 0).reshape(1, 1024)
    d, z, z_mean, z_logvar = _latent(
        hflat, noise,
        wfc_p, fc_b.reshape(1, -1), wml, bml,
        fc_latent_w.T.astype(_BF), fc_latent_b.reshape(1, -1),
        wdec_p, bdec_p, nl)

    g = d.reshape(B, 4, 4, 64)
    g = _dec_conv(_halo(g), _phase_wm(deconv1_w), jnp.tile(deconv1_b, 4),
                  4, 4, 64, relu=True, shuffle=True, out_dtype=_BF)
    g = _dec_conv(_halo(g.reshape(B, 8, 8, 64)), _phase_wm(deconv2_w),
                  jnp.tile(deconv2_b, 4),
                  8, 8, 32, relu=True, shuffle=True, out_dtype=_BF)
    g = _dec_conv(_halo(g.reshape(B, 16, 16, 32)), _phase_wm(deconv3_w),
                  jnp.tile(deconv3_b, 4),
                  16, 16, 16, relu=True, shuffle=True, out_dtype=_BF)
    y4 = _dec_conv(_halo(g.reshape(B, 32, 32, 32)), _phase_wm(deconv4_w),
                   jnp.tile(deconv4_b, 4),
                   32, 32, 8, relu=False, shuffle=False, out_dtype=_BF)
    Y = y4.reshape(B, 33, 33, 2, 2, 3)
    rec = (jnp.transpose(Y, (0, 5, 1, 3, 2, 4)).reshape(B, 3, 66, 66)
           [:, :, 1:65, 1:65])
    return rec.astype(jnp.float32), z, z_mean, z_logvar
